# baseline (speedup 1.0000x reference)
import functools

import jax
import jax.numpy as jnp
from jax import lax
from jax.experimental import pallas as pl
from jax.experimental.pallas import tpu as pltpu
from jax.experimental.pallas import tpu_sc as plsc

NC = 2
NS = 16
L = 16
NW = NC * NS
B = 16384
F = 64
BPW = B // NW
CH = 128
NCH = BPW // CH


def _mf_body(user_r, movie_r, ut_r, mt_r, but_r, bmt_r, bias_r, out_r,
             uidx, midx, urows, mrows, bu, bm, ovec, bvec, sbuf, sem):
    wid = lax.axis_index("s") * NC + lax.axis_index("c")
    base = wid * BPW

    pltpu.sync_copy(user_r.at[pl.ds(wid * NCH, NCH)], uidx)
    pltpu.sync_copy(movie_r.at[pl.ds(wid * NCH, NCH)], midx)
    pltpu.sync_copy(bias_r, bvec)

    copies = []
    for j in range(NCH):
        copies.append(pltpu.async_copy(
            ut_r.at[uidx.at[j]], urows.at[pl.ds(j * CH, CH)], sem))
        copies.append(pltpu.async_copy(
            mt_r.at[midx.at[j]], mrows.at[pl.ds(j * CH, CH)], sem))
        copies.append(pltpu.async_copy(
            but_r.at[uidx.at[j]], bu.at[pl.ds(j * CH, CH)], sem))
        copies.append(pltpu.async_copy(
            bmt_r.at[midx.at[j]], bm.at[pl.ds(j * CH, CH)], sem))
    for c in copies:
        c.wait()

    bias_v = bvec[...]
    lanes = lax.iota(jnp.int32, L)

    def grp(g, carry):
        for r in range(L):
            i = g * L + r
            s = urows[i, pl.ds(0, L)] * mrows[i, pl.ds(0, L)]
            s = s + urows[i, pl.ds(L, L)] * mrows[i, pl.ds(L, L)]
            s = s + urows[i, pl.ds(2 * L, L)] * mrows[i, pl.ds(2 * L, L)]
            s = s + urows[i, pl.ds(3 * L, L)] * mrows[i, pl.ds(3 * L, L)]
            plsc.store_scatter(sbuf, [lanes, jnp.full((L,), r, jnp.int32)], s)
        acc = bu[pl.ds(g * L, L)] + bm[pl.ds(g * L, L)] + carry
        for l in range(L):
            acc = acc + sbuf[l, pl.ds(0, L)]
        ovec[pl.ds(g * L, L)] = acc
        return carry

    lax.fori_loop(0, BPW // L, grp, bias_v)

    pltpu.sync_copy(ovec, out_r.at[pl.ds(base, BPW)])


@functools.partial(
    pl.kernel,
    out_type=jax.ShapeDtypeStruct((B,), jnp.float32),
    mesh=plsc.VectorSubcoreMesh(
        core_axis_name="c", subcore_axis_name="s",
        num_cores=NC, num_subcores=NS),
    compiler_params=pltpu.CompilerParams(
        needs_layout_passes=False, use_tc_tiling_on_sc=False),
    scratch_types=[
        pltpu.VMEM((NCH, CH), jnp.int32),
        pltpu.VMEM((NCH, CH), jnp.int32),
        pltpu.VMEM((BPW, F), jnp.float32),
        pltpu.VMEM((BPW, F), jnp.float32),
        pltpu.VMEM((BPW,), jnp.float32),
        pltpu.VMEM((BPW,), jnp.float32),
        pltpu.VMEM((BPW,), jnp.float32),
        pltpu.VMEM((L,), jnp.float32),
        pltpu.VMEM((L, L), jnp.float32),
        pltpu.SemaphoreType.DMA,
    ],
)
def _mf_kernel(user_r, movie_r, ut_r, mt_r, but_r, bmt_r, bias_r, out_r,
               uidx, midx, urows, mrows, bu, bm, ovec, bvec, sbuf, sem):
    _mf_body(user_r, movie_r, ut_r, mt_r, but_r, bmt_r, bias_r, out_r,
             uidx, midx, urows, mrows, bu, bm, ovec, bvec, sbuf, sem)


@jax.jit
def kernel(user, movie, user_table, movie_table, bias_user_table,
           bias_movie_table, bias):
    user2 = user.astype(jnp.int32).reshape(B // CH, CH)
    movie2 = movie.astype(jnp.int32).reshape(B // CH, CH)
    bias16 = jnp.broadcast_to(bias.astype(jnp.float32), (L,))
    return _mf_kernel(user2, movie2, user_table, movie_table,
                      bias_user_table.reshape(-1),
                      bias_movie_table.reshape(-1), bias16)
